# Initial kernel scaffold; baseline (speedup 1.0000x reference)
import functools

import jax
import jax.numpy as jnp
from jax import lax
from jax.experimental import pallas as pl
from jax.experimental.pallas import tpu as pltpu
from jax.experimental.pallas import tpu_sc as plsc

NC = 2
NS = 16
NW = NC * NS
LW = 16

CH = 80
ZR = 125


def _mesh():
    return plsc.VectorSubcoreMesh(core_axis_name="c", subcore_axis_name="s")


def _degrees_sc(src, dst, n):
    e = src.shape[0]
    ept = e // NW
    nch = ept // CH
    npt = n // NS

    def body(src_hbm, dst_hbm, out_hbm, sbuf, dbuf, ones, zbuf, acc_s, acc_d):
        c = lax.axis_index("c")
        s = lax.axis_index("s")
        wid = s * NC + c

        def fill(i, carry):
            ones[i] = jnp.ones((LW,), jnp.float32)
            return carry

        lax.fori_loop(0, CH, fill, 0)

        def zfill(i, carry):
            zbuf[i] = jnp.zeros((LW,), jnp.float32)
            return carry

        lax.fori_loop(0, ZR, zfill, 0)

        r0 = s * npt

        def zrow(i, carry):
            pltpu.sync_copy(zbuf, acc_s.at[pl.ds(r0 + i * ZR, ZR)])
            pltpu.sync_copy(zbuf, acc_d.at[pl.ds(r0 + i * ZR, ZR)])
            return carry

        lax.fori_loop(0, npt // ZR, zrow, 0)
        plsc.subcore_barrier()

        e0 = wid * ept

        def chunk(g, carry):
            base = e0 + g * CH
            pltpu.sync_copy(src_hbm.at[pl.ds(base, CH)], sbuf)
            pltpu.sync_copy(dst_hbm.at[pl.ds(base, CH)], dbuf)
            pltpu.sync_copy(ones, acc_s.at[sbuf], add=True)
            pltpu.sync_copy(ones, acc_d.at[dbuf], add=True)
            return carry

        lax.fori_loop(0, nch, chunk, 0)
        plsc.subcore_barrier()

        pltpu.sync_copy(acc_s.at[pl.ds(r0, npt)], out_hbm.at[c, 0, pl.ds(r0, npt)])
        pltpu.sync_copy(acc_d.at[pl.ds(r0, npt)], out_hbm.at[c, 1, pl.ds(r0, npt)])

    f = pl.kernel(
        body,
        out_type=jax.ShapeDtypeStruct((NC, 2, n, LW), jnp.float32),
        mesh=_mesh(),
        scratch_types=[
            pltpu.VMEM((CH,), jnp.int32),
            pltpu.VMEM((CH,), jnp.int32),
            pltpu.VMEM((CH, LW), jnp.float32),
            pltpu.VMEM((ZR, LW), jnp.float32),
            pltpu.VMEM_SHARED((n, LW), jnp.float32),
            pltpu.VMEM_SHARED((n, LW), jnp.float32),
        ],
    )
    return f(src, dst)


def _aggregate_sc(t, src, dst, n, d):
    e = src.shape[0]
    ept = e // NW
    nch = ept // CH
    npt = n // NS

    def body(t_hbm, src_hbm, dst_hbm, out_hbm, sbuf, dbuf, rows, zbuf, acc, sem):
        c = lax.axis_index("c")
        s = lax.axis_index("s")
        wid = s * NC + c

        def zfill(i, carry):
            for j in range(d // LW):
                zbuf[i, pl.ds(j * LW, LW)] = jnp.zeros((LW,), jnp.float32)
            return carry

        lax.fori_loop(0, ZR, zfill, 0)

        r0 = s * npt

        def zrow(i, carry):
            pltpu.sync_copy(zbuf, acc.at[pl.ds(r0 + i * ZR, ZR)])
            return carry

        lax.fori_loop(0, npt // ZR, zrow, 0)
        plsc.subcore_barrier()

        e0 = wid * ept

        def chunk(g, carry):
            base = e0 + g * CH
            pltpu.sync_copy(src_hbm.at[pl.ds(base, CH)], sbuf)
            pltpu.sync_copy(dst_hbm.at[pl.ds(base, CH)], dbuf)
            pltpu.async_copy(t_hbm.at[sbuf], rows, sem).wait()
            pltpu.sync_copy(rows, acc.at[dbuf], add=True)
            return carry

        lax.fori_loop(0, nch, chunk, 0)
        plsc.subcore_barrier()

        pltpu.sync_copy(acc.at[pl.ds(r0, npt)], out_hbm.at[c, pl.ds(r0, npt)])

    f = pl.kernel(
        body,
        out_type=jax.ShapeDtypeStruct((NC, n, d), jnp.float32),
        mesh=_mesh(),
        scratch_types=[
            pltpu.VMEM((CH,), jnp.int32),
            pltpu.VMEM((CH,), jnp.int32),
            pltpu.VMEM((CH, d), jnp.float32),
            pltpu.VMEM((ZR, d), jnp.float32),
            pltpu.VMEM_SHARED((n, d), jnp.float32),
            pltpu.SemaphoreType.DMA,
        ],
    )
    return f(t, src, dst)


def _k1_body(x_ref, w1_ref, degp_ref, t1_ref, ns_ref, nd_ref):
    dp = degp_ref[...]
    deg_out = dp[0, 0, :, 0] + dp[1, 0, :, 0]
    deg_in = dp[0, 1, :, 0] + dp[1, 1, :, 0]
    ns = jnp.where(deg_out > 0, lax.rsqrt(jnp.maximum(deg_out, 1.0)), 0.0)
    nd = jnp.where(deg_in > 0, lax.rsqrt(jnp.maximum(deg_in, 1.0)), 0.0)
    t1 = jnp.dot(x_ref[...], w1_ref[...], preferred_element_type=jnp.float32)
    t1_ref[...] = t1 * ns[:, None]
    ns_ref[...] = ns[:, None]
    nd_ref[...] = nd[:, None]


def _k3_body(ap_ref, nd_ref, b1_ref, w2_ref, ns_ref, t2_ref):
    a = ap_ref[0] + ap_ref[1]
    h = jnp.maximum(a * nd_ref[...] + b1_ref[...], 0.0)
    t2 = jnp.dot(h, w2_ref[...], preferred_element_type=jnp.float32)
    t2_ref[...] = t2 * ns_ref[...]


def _k5_body(ap_ref, nd_ref, b2_ref, o_ref):
    a = ap_ref[0] + ap_ref[1]
    o_ref[...] = a * nd_ref[...] + b2_ref[...]


def kernel(x, edge_index, W1, b1, W2, b2):
    n, d_in = x.shape
    d_h = W1.shape[1]
    d_out = W2.shape[1]
    src = edge_index[0]
    dst = edge_index[1]

    degp = _degrees_sc(src, dst, n)

    R = 1000
    grid = (n // R,)

    t1, nsrc, ndst = pl.pallas_call(
        _k1_body,
        grid=grid,
        in_specs=[
            pl.BlockSpec((R, d_in), lambda i: (i, 0)),
            pl.BlockSpec((d_in, d_h), lambda i: (0, 0)),
            pl.BlockSpec((NC, 2, R, LW), lambda i: (0, 0, i, 0)),
        ],
        out_specs=[
            pl.BlockSpec((R, d_h), lambda i: (i, 0)),
            pl.BlockSpec((R, 1), lambda i: (i, 0)),
            pl.BlockSpec((R, 1), lambda i: (i, 0)),
        ],
        out_shape=[
            jax.ShapeDtypeStruct((n, d_h), jnp.float32),
            jax.ShapeDtypeStruct((n, 1), jnp.float32),
            jax.ShapeDtypeStruct((n, 1), jnp.float32),
        ],
    )(x, W1, degp)

    agg1 = _aggregate_sc(t1, src, dst, n, d_h)

    t2 = pl.pallas_call(
        _k3_body,
        grid=grid,
        in_specs=[
            pl.BlockSpec((NC, R, d_h), lambda i: (0, i, 0)),
            pl.BlockSpec((R, 1), lambda i: (i, 0)),
            pl.BlockSpec((1, d_h), lambda i: (0, 0)),
            pl.BlockSpec((d_h, d_out), lambda i: (0, 0)),
            pl.BlockSpec((R, 1), lambda i: (i, 0)),
        ],
        out_specs=pl.BlockSpec((R, d_out), lambda i: (i, 0)),
        out_shape=jax.ShapeDtypeStruct((n, d_out), jnp.float32),
    )(agg1, ndst, b1[None, :], W2, nsrc)

    agg2 = _aggregate_sc(t2, src, dst, n, d_out)

    out = pl.pallas_call(
        _k5_body,
        grid=grid,
        in_specs=[
            pl.BlockSpec((NC, R, d_out), lambda i: (0, i, 0)),
            pl.BlockSpec((R, 1), lambda i: (i, 0)),
            pl.BlockSpec((1, d_out), lambda i: (0, 0)),
        ],
        out_specs=pl.BlockSpec((R, d_out), lambda i: (i, 0)),
        out_shape=jax.ShapeDtypeStruct((n, d_out), jnp.float32),
    )(agg2, ndst, b2[None, :])

    return out


# speedup vs baseline: 5.0386x; 5.0386x over previous
import functools

import jax
import jax.numpy as jnp
from jax import lax
from jax.experimental import pallas as pl
from jax.experimental.pallas import tpu as pltpu
from jax.experimental.pallas import tpu_sc as plsc

NC = 2
NS = 16
NW = NC * NS
LW = 16

CH = 80
ZR = 128


def _npad(n):
    return ((n + 2047) // 2048) * 2048


def _mesh():
    return plsc.VectorSubcoreMesh(core_axis_name="c", subcore_axis_name="s")


def _degrees_sc(src, dst, n):
    e = src.shape[0]
    ept = e // NW
    nch = ept // CH
    npad = _npad(n)
    npt = npad // NS

    def body(src_hbm, dst_hbm, out_hbm, sbuf, dbuf, ones, zbuf, acc_s, acc_d):
        c = lax.axis_index("c")
        s = lax.axis_index("s")
        wid = s * NC + c

        def fill(i, carry):
            ones[i] = jnp.ones((LW,), jnp.float32)
            return carry

        lax.fori_loop(0, CH, fill, 0)

        def zfill(i, carry):
            zbuf[i] = jnp.zeros((LW,), jnp.float32)
            return carry

        lax.fori_loop(0, ZR, zfill, 0)

        r0 = s * npt

        def zrow(i, carry):
            pltpu.sync_copy(zbuf, acc_s.at[pl.ds(r0 + i * ZR, ZR)])
            pltpu.sync_copy(zbuf, acc_d.at[pl.ds(r0 + i * ZR, ZR)])
            return carry

        lax.fori_loop(0, npt // ZR, zrow, 0)
        plsc.subcore_barrier()

        e0 = wid * ept

        def chunk(g, carry):
            base = e0 + g * CH
            pltpu.sync_copy(src_hbm.at[pl.ds(base, CH)], sbuf)
            pltpu.sync_copy(dst_hbm.at[pl.ds(base, CH)], dbuf)
            pltpu.sync_copy(ones, acc_s.at[sbuf], add=True)
            pltpu.sync_copy(ones, acc_d.at[dbuf], add=True)
            return carry

        lax.fori_loop(0, nch, chunk, 0)
        plsc.subcore_barrier()

        pltpu.sync_copy(acc_s.at[pl.ds(r0, npt)], out_hbm.at[c, 0, pl.ds(r0, npt)])
        pltpu.sync_copy(acc_d.at[pl.ds(r0, npt)], out_hbm.at[c, 1, pl.ds(r0, npt)])

    f = pl.kernel(
        body,
        out_type=jax.ShapeDtypeStruct((NC, 2, npad, LW), jnp.float32),
        mesh=_mesh(),
        compiler_params=pltpu.CompilerParams(use_tc_tiling_on_sc=False),
        scratch_types=[
            pltpu.VMEM((CH,), jnp.int32),
            pltpu.VMEM((CH,), jnp.int32),
            pltpu.VMEM((CH, LW), jnp.float32),
            pltpu.VMEM((ZR, LW), jnp.float32),
            pltpu.VMEM_SHARED((npad, LW), jnp.float32),
            pltpu.VMEM_SHARED((npad, LW), jnp.float32),
        ],
    )
    return f(src, dst)


def _aggregate_sc(t, src, dst, n, d):
    e = src.shape[0]
    ept = e // NW
    nch = ept // CH
    npad = _npad(n)
    npt = npad // NS

    def body(t_hbm, src_hbm, dst_hbm, out_hbm, sbuf, dbuf, rows, zbuf, acc, sem):
        c = lax.axis_index("c")
        s = lax.axis_index("s")
        wid = s * NC + c

        def zfill(i, carry):
            for j in range(d // LW):
                zbuf[i, pl.ds(j * LW, LW)] = jnp.zeros((LW,), jnp.float32)
            return carry

        lax.fori_loop(0, ZR, zfill, 0)

        r0 = s * npt

        def zrow(i, carry):
            pltpu.sync_copy(zbuf, acc.at[pl.ds(r0 + i * ZR, ZR)])
            return carry

        lax.fori_loop(0, npt // ZR, zrow, 0)
        plsc.subcore_barrier()

        e0 = wid * ept

        def chunk(g, carry):
            base = e0 + g * CH
            pltpu.sync_copy(src_hbm.at[pl.ds(base, CH)], sbuf)
            pltpu.sync_copy(dst_hbm.at[pl.ds(base, CH)], dbuf)
            pltpu.async_copy(t_hbm.at[sbuf], rows, sem).wait()
            pltpu.sync_copy(rows, acc.at[dbuf], add=True)
            return carry

        lax.fori_loop(0, nch, chunk, 0)
        plsc.subcore_barrier()

        pltpu.sync_copy(acc.at[pl.ds(r0, npt)], out_hbm.at[c, pl.ds(r0, npt)])

    f = pl.kernel(
        body,
        out_type=jax.ShapeDtypeStruct((NC, npad, d), jnp.float32),
        mesh=_mesh(),
        compiler_params=pltpu.CompilerParams(use_tc_tiling_on_sc=False),
        scratch_types=[
            pltpu.VMEM((CH,), jnp.int32),
            pltpu.VMEM((CH,), jnp.int32),
            pltpu.VMEM((CH, d), jnp.float32),
            pltpu.VMEM((ZR, d), jnp.float32),
            pltpu.VMEM_SHARED((npad, d), jnp.float32),
            pltpu.SemaphoreType.DMA,
        ],
    )
    return f(t, src, dst)


def _k1_body(x_ref, w1_ref, degp_ref, t1_ref, ns_ref, nd_ref):
    dp = degp_ref[...]
    deg_out = dp[0, 0, :, 0] + dp[1, 0, :, 0]
    deg_in = dp[0, 1, :, 0] + dp[1, 1, :, 0]
    ns = jnp.where(deg_out > 0, lax.rsqrt(jnp.maximum(deg_out, 1.0)), 0.0)
    nd = jnp.where(deg_in > 0, lax.rsqrt(jnp.maximum(deg_in, 1.0)), 0.0)
    t1 = jnp.dot(x_ref[...], w1_ref[...], preferred_element_type=jnp.float32)
    t1_ref[...] = t1 * ns[:, None]
    ns_ref[...] = ns[:, None]
    nd_ref[...] = nd[:, None]


def _k3_body(ap_ref, nd_ref, b1_ref, w2_ref, ns_ref, t2_ref):
    a = ap_ref[0] + ap_ref[1]
    h = jnp.maximum(a * nd_ref[...] + b1_ref[...], 0.0)
    t2 = jnp.dot(h, w2_ref[...], preferred_element_type=jnp.float32)
    t2_ref[...] = t2 * ns_ref[...]


def _k5_body(ap_ref, nd_ref, b2_ref, o_ref):
    a = ap_ref[0] + ap_ref[1]
    o_ref[...] = a * nd_ref[...] + b2_ref[...]


def kernel(x, edge_index, W1, b1, W2, b2):
    n, d_in = x.shape
    d_h = W1.shape[1]
    d_out = W2.shape[1]
    src = edge_index[0]
    dst = edge_index[1]

    degp = _degrees_sc(src, dst, n)

    R = 1000
    grid = (n // R,)

    t1, nsrc, ndst = pl.pallas_call(
        _k1_body,
        grid=grid,
        in_specs=[
            pl.BlockSpec((R, d_in), lambda i: (i, 0)),
            pl.BlockSpec((d_in, d_h), lambda i: (0, 0)),
            pl.BlockSpec((NC, 2, R, LW), lambda i: (0, 0, i, 0)),
        ],
        out_specs=[
            pl.BlockSpec((R, d_h), lambda i: (i, 0)),
            pl.BlockSpec((R, 1), lambda i: (i, 0)),
            pl.BlockSpec((R, 1), lambda i: (i, 0)),
        ],
        out_shape=[
            jax.ShapeDtypeStruct((n, d_h), jnp.float32),
            jax.ShapeDtypeStruct((n, 1), jnp.float32),
            jax.ShapeDtypeStruct((n, 1), jnp.float32),
        ],
    )(x, W1, degp)

    agg1 = _aggregate_sc(t1, src, dst, n, d_h)

    t2 = pl.pallas_call(
        _k3_body,
        grid=grid,
        in_specs=[
            pl.BlockSpec((NC, R, d_h), lambda i: (0, i, 0)),
            pl.BlockSpec((R, 1), lambda i: (i, 0)),
            pl.BlockSpec((1, d_h), lambda i: (0, 0)),
            pl.BlockSpec((d_h, d_out), lambda i: (0, 0)),
            pl.BlockSpec((R, 1), lambda i: (i, 0)),
        ],
        out_specs=pl.BlockSpec((R, d_out), lambda i: (i, 0)),
        out_shape=jax.ShapeDtypeStruct((n, d_out), jnp.float32),
    )(agg1, ndst, b1[None, :], W2, nsrc)

    agg2 = _aggregate_sc(t2, src, dst, n, d_out)

    out = pl.pallas_call(
        _k5_body,
        grid=grid,
        in_specs=[
            pl.BlockSpec((NC, R, d_out), lambda i: (0, i, 0)),
            pl.BlockSpec((R, 1), lambda i: (i, 0)),
            pl.BlockSpec((1, d_out), lambda i: (0, 0)),
        ],
        out_specs=pl.BlockSpec((R, d_out), lambda i: (i, 0)),
        out_shape=jax.ShapeDtypeStruct((n, d_out), jnp.float32),
    )(agg2, ndst, b2[None, :])

    return out


# speedup vs baseline: 12.0531x; 2.3922x over previous
import functools

import jax
import jax.numpy as jnp
from jax import lax
from jax.experimental import pallas as pl
from jax.experimental.pallas import tpu as pltpu
from jax.experimental.pallas import tpu_sc as plsc

NC = 2
NS = 16
NW = NC * NS
LW = 16

CH = 80
def _npad(n):
    return ((n + 2047) // 2048) * 2048


def _mesh():
    return plsc.VectorSubcoreMesh(core_axis_name="c", subcore_axis_name="s")


def _degrees_sc(src, dst, n):
    nch = src.shape[1]
    npad = _npad(n)
    npt = npad // NS
    assert nch % 2 == 1

    def body(src_hbm, dst_hbm, out_hbm, sidx, didx, ones, zbuf, acc_s, acc_d,
             ss0, ss1, sd0, sd1):
        c = lax.axis_index("c")
        s = lax.axis_index("s")
        wid = s * NC + c

        pltpu.sync_copy(src_hbm.at[wid], sidx)
        pltpu.sync_copy(dst_hbm.at[wid], didx)

        def fill(i, carry):
            ones[i] = jnp.ones((LW,), jnp.float32)
            return carry

        lax.fori_loop(0, CH, fill, 0)

        def zfill(i, carry):
            zbuf[i] = jnp.zeros((LW,), jnp.float32)
            return carry

        lax.fori_loop(0, CH, zfill, 0)

        r0 = s * npt

        def zrow(i, carry):
            pltpu.sync_copy(zbuf, acc_s.at[pl.ds(r0 + i * CH, CH)])
            pltpu.sync_copy(zbuf, acc_d.at[pl.ds(r0 + i * CH, CH)])
            return carry

        lax.fori_loop(0, npt // CH, zrow, 0)
        plsc.subcore_barrier()

        ssem = (ss0, ss1)
        dsem = (sd0, sd1)

        def fire(gi, b):
            pltpu.async_copy(ones, acc_s.at[sidx.at[gi]], ssem[b], add=True)
            pltpu.async_copy(ones, acc_d.at[didx.at[gi]], dsem[b], add=True)

        def wait(gi, b):
            pltpu.make_async_copy(ones, acc_s.at[sidx.at[gi]], ssem[b]).wait()
            pltpu.make_async_copy(ones, acc_d.at[didx.at[gi]], dsem[b]).wait()

        fire(0, 0)
        fire(1, 1)

        def pair(gg, carry):
            for b in (0, 1):
                gi = 2 * gg + b
                wait(gi, b)
                fire(gi + 2, b)
            return carry

        lax.fori_loop(0, (nch - 3) // 2, pair, 0)
        wait(nch - 3, 0)
        fire(nch - 1, 0)
        wait(nch - 2, 1)
        wait(nch - 1, 0)
        plsc.subcore_barrier()

        pltpu.sync_copy(acc_s.at[pl.ds(r0, npt)], out_hbm.at[c, 0, pl.ds(r0, npt)])
        pltpu.sync_copy(acc_d.at[pl.ds(r0, npt)], out_hbm.at[c, 1, pl.ds(r0, npt)])

    f = pl.kernel(
        body,
        out_type=jax.ShapeDtypeStruct((NC, 2, npad, LW), jnp.float32),
        mesh=_mesh(),
        compiler_params=pltpu.CompilerParams(use_tc_tiling_on_sc=False),
        scratch_types=[
            pltpu.VMEM((nch, CH), jnp.int32),
            pltpu.VMEM((nch, CH), jnp.int32),
            pltpu.VMEM((CH, LW), jnp.float32),
            pltpu.VMEM((CH, LW), jnp.float32),
            pltpu.VMEM_SHARED((npad, LW), jnp.float32),
            pltpu.VMEM_SHARED((npad, LW), jnp.float32),
            pltpu.SemaphoreType.DMA,
            pltpu.SemaphoreType.DMA,
            pltpu.SemaphoreType.DMA,
            pltpu.SemaphoreType.DMA,
        ],
    )
    return f(src, dst)


def _aggregate_sc(t, src, dst, n, d):
    nch = src.shape[1]
    npad = _npad(n)
    npt = npad // NS
    assert nch % 2 == 1

    def body(t_hbm, src_hbm, dst_hbm, out_hbm, sidx, didx, rows0, rows1,
             acc, sg0, sg1):
        c = lax.axis_index("c")
        s = lax.axis_index("s")
        wid = s * NC + c

        pltpu.sync_copy(src_hbm.at[wid], sidx)
        pltpu.sync_copy(dst_hbm.at[wid], didx)

        def zfill(i, carry):
            for j in range(d // LW):
                rows0[i, pl.ds(j * LW, LW)] = jnp.zeros((LW,), jnp.float32)
            return carry

        lax.fori_loop(0, CH, zfill, 0)

        r0 = s * npt

        def zrow(i, carry):
            pltpu.sync_copy(rows0, acc.at[pl.ds(r0 + i * CH, CH)])
            return carry

        lax.fori_loop(0, npt // CH, zrow, 0)
        plsc.subcore_barrier()

        rows = (rows0, rows1)
        gsem = (sg0, sg1)

        def fire(gi, b):
            pltpu.async_copy(t_hbm.at[sidx.at[gi]], rows[b], gsem[b])

        def wait(gi, b):
            pltpu.make_async_copy(t_hbm.at[sidx.at[gi]], rows[b], gsem[b]).wait()

        def scat(gi, b):
            pltpu.sync_copy(rows[b], acc.at[didx.at[gi]], add=True)

        fire(0, 0)
        fire(1, 1)

        def pair(gg, carry):
            for b in (0, 1):
                gi = 2 * gg + b
                wait(gi, b)
                scat(gi, b)
                fire(gi + 2, b)
            return carry

        lax.fori_loop(0, (nch - 3) // 2, pair, 0)
        wait(nch - 3, 0)
        scat(nch - 3, 0)
        fire(nch - 1, 0)
        wait(nch - 2, 1)
        scat(nch - 2, 1)
        wait(nch - 1, 0)
        scat(nch - 1, 0)
        plsc.subcore_barrier()

        pltpu.sync_copy(acc.at[pl.ds(r0, npt)], out_hbm.at[c, pl.ds(r0, npt)])

    f = pl.kernel(
        body,
        out_type=jax.ShapeDtypeStruct((NC, npad, d), jnp.float32),
        mesh=_mesh(),
        compiler_params=pltpu.CompilerParams(use_tc_tiling_on_sc=False),
        scratch_types=[
            pltpu.VMEM((nch, CH), jnp.int32),
            pltpu.VMEM((nch, CH), jnp.int32),
            pltpu.VMEM((CH, d), jnp.float32),
            pltpu.VMEM((CH, d), jnp.float32),
            pltpu.VMEM_SHARED((npad, d), jnp.float32),
            pltpu.SemaphoreType.DMA,
            pltpu.SemaphoreType.DMA,
        ],
    )
    return f(t, src, dst)


def _k1_body(x_ref, w1_ref, degp_ref, t1_ref, ns_ref, nd_ref):
    dp = degp_ref[...]
    deg_out = dp[0, 0, :, 0] + dp[1, 0, :, 0]
    deg_in = dp[0, 1, :, 0] + dp[1, 1, :, 0]
    ns = jnp.where(deg_out > 0, lax.rsqrt(jnp.maximum(deg_out, 1.0)), 0.0)
    nd = jnp.where(deg_in > 0, lax.rsqrt(jnp.maximum(deg_in, 1.0)), 0.0)
    t1 = jnp.dot(x_ref[...], w1_ref[...], preferred_element_type=jnp.float32)
    t1_ref[...] = t1 * ns[:, None]
    ns_ref[...] = ns[:, None]
    nd_ref[...] = nd[:, None]


def _k3_body(ap_ref, nd_ref, b1_ref, w2_ref, ns_ref, t2_ref):
    a = ap_ref[0] + ap_ref[1]
    h = jnp.maximum(a * nd_ref[...] + b1_ref[...], 0.0)
    t2 = jnp.dot(h, w2_ref[...], preferred_element_type=jnp.float32)
    t2_ref[...] = t2 * ns_ref[...]


def _k5_body(ap_ref, nd_ref, b2_ref, o_ref):
    a = ap_ref[0] + ap_ref[1]
    o_ref[...] = a * nd_ref[...] + b2_ref[...]


def kernel(x, edge_index, W1, b1, W2, b2):
    n, d_in = x.shape
    d_h = W1.shape[1]
    d_out = W2.shape[1]
    e = edge_index.shape[1]
    nch = e // NW // CH
    er = edge_index.reshape(2, NW, nch, CH)
    src = er[0]
    dst = er[1]

    degp = _degrees_sc(src, dst, n)

    R = 1000
    grid = (n // R,)

    t1, nsrc, ndst = pl.pallas_call(
        _k1_body,
        grid=grid,
        in_specs=[
            pl.BlockSpec((R, d_in), lambda i: (i, 0)),
            pl.BlockSpec((d_in, d_h), lambda i: (0, 0)),
            pl.BlockSpec((NC, 2, R, LW), lambda i: (0, 0, i, 0)),
        ],
        out_specs=[
            pl.BlockSpec((R, d_h), lambda i: (i, 0)),
            pl.BlockSpec((R, 1), lambda i: (i, 0)),
            pl.BlockSpec((R, 1), lambda i: (i, 0)),
        ],
        out_shape=[
            jax.ShapeDtypeStruct((n, d_h), jnp.float32),
            jax.ShapeDtypeStruct((n, 1), jnp.float32),
            jax.ShapeDtypeStruct((n, 1), jnp.float32),
        ],
    )(x, W1, degp)

    agg1 = _aggregate_sc(t1, src, dst, n, d_h)

    t2 = pl.pallas_call(
        _k3_body,
        grid=grid,
        in_specs=[
            pl.BlockSpec((NC, R, d_h), lambda i: (0, i, 0)),
            pl.BlockSpec((R, 1), lambda i: (i, 0)),
            pl.BlockSpec((1, d_h), lambda i: (0, 0)),
            pl.BlockSpec((d_h, d_out), lambda i: (0, 0)),
            pl.BlockSpec((R, 1), lambda i: (i, 0)),
        ],
        out_specs=pl.BlockSpec((R, d_out), lambda i: (i, 0)),
        out_shape=jax.ShapeDtypeStruct((n, d_out), jnp.float32),
    )(agg1, ndst, b1[None, :], W2, nsrc)

    agg2 = _aggregate_sc(t2, src, dst, n, d_out)

    out = pl.pallas_call(
        _k5_body,
        grid=grid,
        in_specs=[
            pl.BlockSpec((NC, R, d_out), lambda i: (0, i, 0)),
            pl.BlockSpec((R, 1), lambda i: (i, 0)),
            pl.BlockSpec((1, d_out), lambda i: (0, 0)),
        ],
        out_specs=pl.BlockSpec((R, d_out), lambda i: (i, 0)),
        out_shape=jax.ShapeDtypeStruct((n, d_out), jnp.float32),
    )(agg2, ndst, b2[None, :])

    return out


# speedup vs baseline: 13.0655x; 1.0840x over previous
import functools

import jax
import jax.numpy as jnp
from jax import lax
from jax.experimental import pallas as pl
from jax.experimental.pallas import tpu as pltpu
from jax.experimental.pallas import tpu_sc as plsc

NC = 2
NS = 16
NW = NC * NS
LW = 16

CH = 80
ACH = 40
RB = 5
def _npad(n):
    return ((n + 2047) // 2048) * 2048


def _mesh():
    return plsc.VectorSubcoreMesh(core_axis_name="c", subcore_axis_name="s")


def _degrees_sc(src, dst, n):
    nch = src.shape[1]
    npad = _npad(n)
    npt = npad // NS

    def body(src_hbm, dst_hbm, out_hbm, sidx, didx, ones_s, ones_d, zbuf, acc,
             ss0, ss1, sd0, sd1):
        c = lax.axis_index("c")
        s = lax.axis_index("s")
        wid = s * NC + c

        pltpu.sync_copy(src_hbm.at[wid], sidx)
        pltpu.sync_copy(dst_hbm.at[wid], didx)

        lane = lax.iota(jnp.int32, 16)
        one = jnp.ones((LW,), jnp.float32)
        zero = jnp.zeros((LW,), jnp.float32)

        def fill(i, carry):
            ones_s[i] = jnp.where(lane < 8, one, zero)
            ones_d[i] = jnp.where(lane < 8, zero, one)
            zbuf[i] = zero
            return carry

        lax.fori_loop(0, CH, fill, 0)

        r0 = s * npt

        def zrow(i, carry):
            pltpu.sync_copy(zbuf, acc.at[pl.ds(r0 + i * CH, CH)])
            return carry

        lax.fori_loop(0, npt // CH, zrow, 0)
        plsc.subcore_barrier()

        ssem = (ss0, ss1)
        dsem = (sd0, sd1)

        def fire(gi, b):
            pltpu.async_copy(ones_s, acc.at[sidx.at[gi]], ssem[b], add=True)
            pltpu.async_copy(ones_d, acc.at[didx.at[gi]], dsem[b], add=True)

        def wait(gi, b):
            pltpu.make_async_copy(ones_s, acc.at[sidx.at[gi]], ssem[b]).wait()
            pltpu.make_async_copy(ones_d, acc.at[didx.at[gi]], dsem[b]).wait()

        fire(0, 0)
        fire(1, 1)

        lp = (nch - 2) // 2

        def pair(gg, carry):
            for b in (0, 1):
                gi = 2 * gg + b
                wait(gi, b)
                fire(gi + 2, b)
            return carry

        lax.fori_loop(0, lp, pair, 0)
        for gi in range(2 * lp, nch):
            b = gi % 2
            wait(gi, b)
            if gi + 2 < nch:
                fire(gi + 2, b)
        plsc.subcore_barrier()

        pltpu.sync_copy(acc.at[pl.ds(r0, npt)], out_hbm.at[c, pl.ds(r0, npt)])

    f = pl.kernel(
        body,
        out_type=jax.ShapeDtypeStruct((NC, npad, LW), jnp.float32),
        mesh=_mesh(),
        compiler_params=pltpu.CompilerParams(use_tc_tiling_on_sc=False),
        scratch_types=[
            pltpu.VMEM((nch, CH), jnp.int32),
            pltpu.VMEM((nch, CH), jnp.int32),
            pltpu.VMEM((CH, LW), jnp.float32),
            pltpu.VMEM((CH, LW), jnp.float32),
            pltpu.VMEM((CH, LW), jnp.float32),
            pltpu.VMEM_SHARED((npad, LW), jnp.float32),
            pltpu.SemaphoreType.DMA,
            pltpu.SemaphoreType.DMA,
            pltpu.SemaphoreType.DMA,
            pltpu.SemaphoreType.DMA,
        ],
    )
    return f(src, dst)


def _aggregate_sc(t, src, dst, n, d):
    nch = src.shape[1]
    ch = src.shape[2]
    npad = _npad(n)
    npt = npad // NS
    assert nch % RB == 0 and npt % ch == 0

    def body(t_hbm, src_hbm, dst_hbm, out_hbm, sidx, didx, rows, acc, gsems, ssems):
        c = lax.axis_index("c")
        s = lax.axis_index("s")
        wid = s * NC + c

        pltpu.sync_copy(src_hbm.at[wid], sidx)
        pltpu.sync_copy(dst_hbm.at[wid], didx)

        def zfill(i, carry):
            for j in range(d // LW):
                rows[0][i, pl.ds(j * LW, LW)] = jnp.zeros((LW,), jnp.float32)
            return carry

        lax.fori_loop(0, ch, zfill, 0)

        r0 = s * npt

        def zrow(i, carry):
            pltpu.sync_copy(rows[0], acc.at[pl.ds(r0 + i * ch, ch)])
            return carry

        lax.fori_loop(0, npt // ch, zrow, 0)
        plsc.subcore_barrier()

        def fire_g(gi, b):
            pltpu.async_copy(t_hbm.at[sidx.at[gi]], rows[b], gsems[b])

        def wait_g(gi, b):
            pltpu.make_async_copy(t_hbm.at[sidx.at[gi]], rows[b], gsems[b]).wait()

        def fire_s(gi, b):
            pltpu.async_copy(rows[b], acc.at[didx.at[gi]], ssems[b], add=True)

        def wait_s(gi, b):
            pltpu.make_async_copy(rows[b], acc.at[didx.at[gi]], ssems[b]).wait()

        for b in range(RB):
            fire_g(b, b)

        def grp(gg, carry):
            g = RB * gg
            for b in range(RB):
                wait_g(g + b, b)
                fire_s(g + b, b)
            for b in range(RB):
                wait_s(g + b, b)
                fire_g(g + RB + b, b)
            return carry

        lax.fori_loop(0, nch // RB - 1, grp, 0)
        ge = nch - RB
        for b in range(RB):
            wait_g(ge + b, b)
            fire_s(ge + b, b)
        for b in range(RB):
            wait_s(ge + b, b)
        plsc.subcore_barrier()

        pltpu.sync_copy(acc.at[pl.ds(r0, npt)], out_hbm.at[c, pl.ds(r0, npt)])

    def wrapped(t_hbm, src_hbm, dst_hbm, out_hbm, sidx, didx, *rest):
        rows = rest[:RB]
        acc = rest[RB]
        gsems = rest[RB + 1:2 * RB + 1]
        ssems = rest[2 * RB + 1:]
        return body(t_hbm, src_hbm, dst_hbm, out_hbm, sidx, didx, rows, acc,
                    gsems, ssems)

    f = pl.kernel(
        wrapped,
        out_type=jax.ShapeDtypeStruct((NC, npad, d), jnp.float32),
        mesh=_mesh(),
        compiler_params=pltpu.CompilerParams(use_tc_tiling_on_sc=False),
        scratch_types=[
            pltpu.VMEM((nch, ch), jnp.int32),
            pltpu.VMEM((nch, ch), jnp.int32),
        ] + [pltpu.VMEM((ch, d), jnp.float32) for _ in range(RB)]
          + [pltpu.VMEM_SHARED((npad, d), jnp.float32)]
          + [pltpu.SemaphoreType.DMA for _ in range(2 * RB)],
    )
    return f(t, src, dst)


def _k1_body(x_ref, w1_ref, degp_ref, t1_ref, ns_ref, nd_ref):
    dp = degp_ref[...]
    deg_out = dp[0, :, 0] + dp[1, :, 0]
    deg_in = dp[0, :, 8] + dp[1, :, 8]
    ns = jnp.where(deg_out > 0, lax.rsqrt(jnp.maximum(deg_out, 1.0)), 0.0)
    nd = jnp.where(deg_in > 0, lax.rsqrt(jnp.maximum(deg_in, 1.0)), 0.0)
    t1 = jnp.dot(x_ref[...], w1_ref[...], preferred_element_type=jnp.float32)
    t1_ref[...] = t1 * ns[:, None]
    ns_ref[...] = ns[:, None]
    nd_ref[...] = nd[:, None]


def _k3_body(ap_ref, nd_ref, b1_ref, w2_ref, ns_ref, t2_ref):
    a = ap_ref[0] + ap_ref[1]
    h = jnp.maximum(a * nd_ref[...] + b1_ref[...], 0.0)
    t2 = jnp.dot(h, w2_ref[...], preferred_element_type=jnp.float32)
    t2_ref[...] = t2 * ns_ref[...]


def _k5_body(ap_ref, nd_ref, b2_ref, o_ref):
    a = ap_ref[0] + ap_ref[1]
    o_ref[...] = a * nd_ref[...] + b2_ref[...]


def kernel(x, edge_index, W1, b1, W2, b2):
    n, d_in = x.shape
    d_h = W1.shape[1]
    d_out = W2.shape[1]
    e = edge_index.shape[1]
    erd = edge_index.reshape(2, NW, e // NW // CH, CH)
    era = edge_index.reshape(2, NW, e // NW // ACH, ACH)

    degp = _degrees_sc(erd[0], erd[1], n)

    R = 1000
    grid = (n // R,)

    t1, nsrc, ndst = pl.pallas_call(
        _k1_body,
        grid=grid,
        in_specs=[
            pl.BlockSpec((R, d_in), lambda i: (i, 0)),
            pl.BlockSpec((d_in, d_h), lambda i: (0, 0)),
            pl.BlockSpec((NC, R, LW), lambda i: (0, i, 0)),
        ],
        out_specs=[
            pl.BlockSpec((R, d_h), lambda i: (i, 0)),
            pl.BlockSpec((R, 1), lambda i: (i, 0)),
            pl.BlockSpec((R, 1), lambda i: (i, 0)),
        ],
        out_shape=[
            jax.ShapeDtypeStruct((n, d_h), jnp.float32),
            jax.ShapeDtypeStruct((n, 1), jnp.float32),
            jax.ShapeDtypeStruct((n, 1), jnp.float32),
        ],
    )(x, W1, degp)

    agg1 = _aggregate_sc(t1, era[0], era[1], n, d_h)

    t2 = pl.pallas_call(
        _k3_body,
        grid=grid,
        in_specs=[
            pl.BlockSpec((NC, R, d_h), lambda i: (0, i, 0)),
            pl.BlockSpec((R, 1), lambda i: (i, 0)),
            pl.BlockSpec((1, d_h), lambda i: (0, 0)),
            pl.BlockSpec((d_h, d_out), lambda i: (0, 0)),
            pl.BlockSpec((R, 1), lambda i: (i, 0)),
        ],
        out_specs=pl.BlockSpec((R, d_out), lambda i: (i, 0)),
        out_shape=jax.ShapeDtypeStruct((n, d_out), jnp.float32),
    )(agg1, ndst, b1[None, :], W2, nsrc)

    agg2 = _aggregate_sc(t2, era[0], era[1], n, d_out)

    out = pl.pallas_call(
        _k5_body,
        grid=grid,
        in_specs=[
            pl.BlockSpec((NC, R, d_out), lambda i: (0, i, 0)),
            pl.BlockSpec((R, 1), lambda i: (i, 0)),
            pl.BlockSpec((1, d_out), lambda i: (0, 0)),
        ],
        out_specs=pl.BlockSpec((R, d_out), lambda i: (i, 0)),
        out_shape=jax.ShapeDtypeStruct((n, d_out), jnp.float32),
    )(agg2, ndst, b2[None, :])

    return out
